# baseline (speedup 1.0000x reference)
import functools

import jax
import jax.numpy as jnp
from jax import lax
from jax.experimental import pallas as pl
from jax.experimental.pallas import tpu as pltpu
from jax.experimental.pallas import tpu_sc as plsc

NUM_CORES = 2
NUM_SUBCORES = 16
NW = NUM_CORES * NUM_SUBCORES
LANES = 16
HP = 128
NS = 4


@functools.cache
def _build(B, S, H, V):
    bw = B // NW
    assert B % (NW * LANES) == 0 and S % NS == 0 and H % LANES == 0

    mesh = plsc.VectorSubcoreMesh(
        core_axis_name="c", subcore_axis_name="s",
        num_cores=NUM_CORES, num_subcores=NUM_SUBCORES)

    def body(x_hbm, emb_hbm, pos_hbm, out_hbm, pos_v, *rest):
        idx = rest[:NS]
        bg = rest[NS:2 * NS]
        bt = rest[2 * NS:3 * NS]
        isem = rest[3 * NS:4 * NS]
        gsem = rest[4 * NS:5 * NS]
        osem = rest[5 * NS:6 * NS]

        wid = lax.axis_index("s") * NUM_CORES + lax.axis_index("c")
        col0 = pl.multiple_of(wid * bw, bw)
        pltpu.sync_copy(pos_hbm, pos_v)

        def idx_dma(s, b):
            pltpu.async_copy(
                x_hbm.at[pl.ds(s * B + col0, bw)], idx[b], isem[b])

        def gather(b):
            pltpu.async_copy(emb_hbm.at[idx[b]], bg[b], gsem[b])

        for b in range(NS):
            idx_dma(b, b)
        for b in range(2):
            pltpu.make_async_copy(
                x_hbm.at[pl.ds(0, bw)], idx[b], isem[b]).wait()
            gather(b)

        iota = lax.iota(jnp.int32, LANES)

        def outer(i, _):
            for b in range(NS):
                s = i * NS + b
                b2 = (b + 2) % NS
                pltpu.make_async_copy(emb_hbm.at[idx[b]], bg[b],
                                      gsem[b]).wait()

                @pl.when(s + 2 < S)
                def _():
                    pltpu.make_async_copy(
                        x_hbm.at[pl.ds(0, bw)], idx[b2], isem[b2]).wait()
                    pltpu.async_copy(emb_hbm.at[idx[b2]], bg[b2], gsem[b2])

                @pl.when(s >= NS)
                def _():
                    pltpu.make_async_copy(
                        bt[b], out_hbm.at[0, :, pl.ds(col0, bw)],
                        osem[b]).wait()

                svec = jnp.full((LANES,), s, jnp.int32)

                @plsc.parallel_loop(0, (H // LANES) * LANES, unroll=2)
                def t_body(t):
                    h0 = (t >> 4) * LANES
                    d = t & (LANES - 1)
                    hrot = h0 + ((iota + d) & (LANES - 1))
                    p = plsc.load_gather(pos_v, [svec, hrot])
                    for c in range(bw // LANES):
                        tl = iota + (c * LANES)
                        v = plsc.load_gather(bg[b], [tl, hrot])
                        plsc.store_scatter(bt[b], [hrot, tl], v + p)
                pltpu.async_copy(
                    bt[b], out_hbm.at[s, :, pl.ds(col0, bw)], osem[b])

                @pl.when(s + NS < S)
                def _():
                    idx_dma(s + NS, b)

            return 0

        lax.fori_loop(0, S // NS, outer, 0)
        for b in range(NS):
            pltpu.make_async_copy(
                bt[b], out_hbm.at[0, :, pl.ds(col0, bw)], osem[b]).wait()

    return pl.kernel(
        body,
        out_type=jax.ShapeDtypeStruct((S, H, B), jnp.float32),
        mesh=mesh,
        scratch_types=(
            [pltpu.VMEM((S, H), jnp.float32)]
            + [pltpu.VMEM((bw,), jnp.int32)] * NS
            + [pltpu.VMEM((bw, HP), jnp.float32)] * NS
            + [pltpu.VMEM((H, bw), jnp.float32)] * NS
            + [pltpu.SemaphoreType.DMA] * (3 * NS)
        ),
        compiler_params=pltpu.CompilerParams(needs_layout_passes=False),
    )


def kernel(x, emb_table, pos_table):
    B, S = x.shape
    V, H = emb_table.shape
    embp = jnp.concatenate(
        [emb_table, jnp.zeros((V, HP - H), emb_table.dtype)], axis=1)
    xr = x.T.reshape(B * S).astype(jnp.int32)
    out_shb = _build(B, S, H, V)(xr, embp, pos_table[:S])
    return out_shb.transpose(2, 0, 1)


# speedup vs baseline: 1.0136x; 1.0136x over previous
import functools

import jax
import jax.numpy as jnp
from jax import lax
from jax.experimental import pallas as pl
from jax.experimental.pallas import tpu as pltpu
from jax.experimental.pallas import tpu_sc as plsc

NUM_CORES = 2
NUM_SUBCORES = 16
NW = NUM_CORES * NUM_SUBCORES
LANES = 16
HP = 128
NS = 4


@functools.cache
def _build(B, S, H, V):
    bw = B // NW
    assert B % (NW * LANES) == 0 and S % NS == 0 and H % LANES == 0

    mesh = plsc.VectorSubcoreMesh(
        core_axis_name="c", subcore_axis_name="s",
        num_cores=NUM_CORES, num_subcores=NUM_SUBCORES)

    def body(x_hbm, emb_hbm, pos_hbm, out_hbm, pos_v, *rest):
        idx = rest[:NS]
        bg = rest[NS:2 * NS]
        bt = rest[2 * NS:3 * NS]
        isem = rest[3 * NS:4 * NS]
        gsem = rest[4 * NS:5 * NS]
        osem = rest[5 * NS:6 * NS]

        wid = lax.axis_index("s") * NUM_CORES + lax.axis_index("c")
        col0 = pl.multiple_of(wid * bw, bw)
        pltpu.sync_copy(pos_hbm, pos_v)

        def idx_dma(s, b):
            pltpu.async_copy(
                x_hbm.at[pl.ds(s * B + col0, bw)], idx[b], isem[b])

        def gather(b):
            pltpu.async_copy(emb_hbm.at[idx[b]], bg[b], gsem[b])

        for b in range(NS):
            idx_dma(b, b)
        for b in range(3):
            pltpu.make_async_copy(
                x_hbm.at[pl.ds(0, bw)], idx[b], isem[b]).wait()
            gather(b)

        iota = lax.iota(jnp.int32, LANES)

        def outer(i, _):
            for b in range(NS):
                s = i * NS + b
                b3 = (b + 3) % NS
                pltpu.make_async_copy(emb_hbm.at[idx[b]], bg[b],
                                      gsem[b]).wait()

                @pl.when(s + 3 < S)
                def _():
                    pltpu.make_async_copy(
                        x_hbm.at[pl.ds(0, bw)], idx[b3], isem[b3]).wait()
                    pltpu.async_copy(emb_hbm.at[idx[b3]], bg[b3], gsem[b3])

                @pl.when(s + NS < S)
                def _():
                    idx_dma(s + NS, b)

                @pl.when(s >= NS)
                def _():
                    pltpu.make_async_copy(
                        bt[b], out_hbm.at[0, :, pl.ds(col0, bw)],
                        osem[b]).wait()

                svec = jnp.full((LANES,), s, jnp.int32)

                @plsc.parallel_loop(0, (H // LANES) * LANES, unroll=2)
                def t_body(t):
                    h0 = (t >> 4) * LANES
                    d = t & (LANES - 1)
                    hrot = h0 + ((iota + d) & (LANES - 1))
                    p = plsc.load_gather(pos_v, [svec, hrot])
                    for c in range(bw // LANES):
                        tl = iota + (c * LANES)
                        v = plsc.load_gather(bg[b], [tl, hrot])
                        plsc.store_scatter(bt[b], [hrot, tl], v + p)
                pltpu.async_copy(
                    bt[b], out_hbm.at[s, :, pl.ds(col0, bw)], osem[b])

            return 0

        lax.fori_loop(0, S // NS, outer, 0)
        for b in range(NS):
            pltpu.make_async_copy(
                bt[b], out_hbm.at[0, :, pl.ds(col0, bw)], osem[b]).wait()

    return pl.kernel(
        body,
        out_type=jax.ShapeDtypeStruct((S, H, B), jnp.float32),
        mesh=mesh,
        scratch_types=(
            [pltpu.VMEM((S, H), jnp.float32)]
            + [pltpu.VMEM((bw,), jnp.int32)] * NS
            + [pltpu.VMEM((bw, HP), jnp.float32)] * NS
            + [pltpu.VMEM((H, bw), jnp.float32)] * NS
            + [pltpu.SemaphoreType.DMA] * (3 * NS)
        ),
        compiler_params=pltpu.CompilerParams(needs_layout_passes=False),
    )


def kernel(x, emb_table, pos_table):
    B, S = x.shape
    V, H = emb_table.shape
    embp = jnp.pad(emb_table, ((0, 0), (0, HP - H)))
    xr = x.T.reshape(B * S).astype(jnp.int32)
    out_shb = _build(B, S, H, V)(xr, embp, pos_table[:S])
    return out_shb.transpose(2, 0, 1)
